# baseline (speedup 1.0000x reference)
import functools

import jax
import jax.numpy as jnp
from jax import lax
from jax.experimental import pallas as pl
from jax.experimental.pallas import tpu as pltpu
from jax.experimental.pallas import tpu_sc as plsc

VOCAB = 1000000
DIM = 32
B = 16384
L = 200

NC = 2
NS = 16
LANES = 16

NW = NC * NS
RPW = B // NW
CH = 8
NCHUNK = RPW // CH
LP = 208
NSL = LP // LANES
GI = CH * LP
NDMA = GI // 64


def _body(stacks_hbm, lens_hbm, table_hbm, out_hbm,
          idx_raw, idx_mod, gath, lens_v, out_v, row0_v, sem):
    wid = lax.axis_index("s") * NC + lax.axis_index("c")
    wbase = wid * RPW

    iota = lax.iota(jnp.int32, LANES)

    pltpu.sync_copy(lens_hbm.at[pl.ds(wbase, RPW)], lens_v)
    pltpu.sync_copy(table_hbm.at[pl.ds(0, 1)], row0_v)
    r0a = row0_v[0, pl.ds(0, LANES)]
    r0b = row0_v[0, pl.ds(LANES, LANES)]

    def get_len(lb):
        off = (lb >> 4) << 4
        lane = lb & (LANES - 1)
        v16 = lens_v[pl.ds(off, LANES)]
        return jnp.sum(jnp.where(iota == lane, v16, 0))

    def chunk(c, _):
        base = wbase + c * CH
        pltpu.sync_copy(stacks_hbm.at[pl.ds(base, CH)],
                        idx_raw.at[:, pl.ds(0, L)])

        def mask_row(r, _):
            len_s = get_len(c * CH + r)

            def mask_slice(j, _):
                pos = j * LANES + iota
                idxv = idx_raw[r, pl.ds(j * LANES, LANES)]
                sel = jnp.where(pos < len_s, idxv, 0)
                f = r * LP + j * LANES
                idx_mod[f >> 6, pl.ds(f & 63, LANES)] = sel
                return 0

            lax.fori_loop(0, NSL, mask_slice, 0)
            return 0

        lax.fori_loop(0, CH, mask_row, 0)

        copies = [
            pltpu.async_copy(table_hbm.at[idx_mod.at[j]],
                             gath.at[pl.ds(j * 64, 64)], sem)
            for j in range(NDMA)
        ]
        for cp in copies:
            cp.wait()

        def fin_row(r, _):
            gbase = r * LP

            def acc_body(p, carry):
                a0, a1 = carry
                g = gbase + p
                a0 = a0 + gath[g, pl.ds(0, LANES)]
                a1 = a1 + gath[g, pl.ds(LANES, LANES)]
                return (a0, a1)

            zero = jnp.zeros((LANES,), jnp.float32)
            a0, a1 = lax.fori_loop(0, LP, acc_body, (zero, zero))

            def z_body(j, zc):
                f = gbase + j * LANES
                sel = idx_mod[f >> 6, pl.ds(f & 63, LANES)]
                return zc + jnp.where(sel == 0, 1, 0)

            zc = lax.fori_loop(0, NSL, z_body, jnp.zeros((LANES,), jnp.int32))
            zf = jnp.broadcast_to(jnp.sum(zc), (LANES,)).astype(jnp.float32)

            len_s = get_len(c * CH + r)
            den = jnp.broadcast_to(jnp.maximum(len_s, 1), (LANES,))
            inv = jnp.ones((LANES,), jnp.float32) / den.astype(jnp.float32)
            out_v[r, pl.ds(0, LANES)] = (a0 - zf * r0a) * inv
            out_v[r, pl.ds(LANES, LANES)] = (a1 - zf * r0b) * inv
            return 0

        lax.fori_loop(0, CH, fin_row, 0)
        pltpu.sync_copy(out_v, out_hbm.at[pl.ds(base, CH)])
        return 0

    lax.fori_loop(0, NCHUNK, chunk, 0)


@jax.jit
def _run(stacks, stack_lens, table):
    mesh = plsc.VectorSubcoreMesh(core_axis_name="c", subcore_axis_name="s")
    f = pl.kernel(
        _body,
        out_type=jax.ShapeDtypeStruct((B, DIM), jnp.float32),
        mesh=mesh,
        scratch_types=[
            pltpu.VMEM((CH, LP), jnp.int32),
            pltpu.VMEM((GI // 64, 64), jnp.int32),
            pltpu.VMEM((GI, DIM), jnp.float32),
            pltpu.VMEM((RPW,), jnp.int32),
            pltpu.VMEM((CH, DIM), jnp.float32),
            pltpu.VMEM((1, DIM), jnp.float32),
            pltpu.SemaphoreType.DMA,
        ],
        compiler_params=pltpu.CompilerParams(use_tc_tiling_on_sc=False,
                                            needs_layout_passes=False),
    )
    return f(stacks, stack_lens, table)


def kernel(stacks, stack_lens, table):
    return _run(stacks.astype(jnp.int32), stack_lens.astype(jnp.int32), table)


# speedup vs baseline: 23.6506x; 23.6506x over previous
import jax
import jax.numpy as jnp
from jax import lax
from jax.experimental import pallas as pl
from jax.experimental.pallas import tpu as pltpu
from jax.experimental.pallas import tpu_sc as plsc

VOCAB = 1000000
DIM = 32
B = 16384
L = 200

NC = 2
NS = 16
LANES = 16

NW = NC * NS
RPW = B // NW
CH = 8
NCHUNK = RPW // CH
LP = 208
NSL = LP // LANES
GI = CH * LP
NIDX = 128
NBLK = GI // NIDX


def _body(stacks_hbm, lens_hbm, table_hbm, out_hbm,
          idx_raw, idx_m0, idx_m1, gath0, gath1, lens_v, out_v, row0_v,
          sem0, sem1):
    wid = lax.axis_index("s") * NC + lax.axis_index("c")
    wbase = wid * RPW

    iota = lax.iota(jnp.int32, LANES)
    zero16i = jnp.zeros((LANES,), jnp.int32)

    pltpu.sync_copy(lens_hbm.at[pl.ds(wbase, RPW)], lens_v)
    pltpu.sync_copy(table_hbm.at[pl.ds(0, 1)], row0_v)
    r0a = row0_v[0, pl.ds(0, LANES)]
    r0b = row0_v[0, pl.ds(LANES, LANES)]

    spread = wid * 4096 + iota

    def init_m(i, _):
        v = spread + i * LANES
        idx_m0[i >> 3, pl.ds((i & 7) * LANES, LANES)] = v
        idx_m1[i >> 3, pl.ds((i & 7) * LANES, LANES)] = v
        return 0

    lax.fori_loop(0, NBLK * (NIDX // LANES), init_m, 0)
    for r in range(CH):
        idx_raw[r, pl.ds(L - 8, LANES)] = spread + (GI + r * LANES)

    def get_len(lb):
        off = (lb >> 4) << 4
        lane = lb & (LANES - 1)
        v16 = lens_v[pl.ds(off, LANES)]
        return jnp.sum(jnp.where(iota == lane, v16, 0))

    def prep(c, idx_m, gath, sem):
        base = wbase + c * CH
        pltpu.sync_copy(stacks_hbm.at[pl.ds(base, CH)],
                        idx_raw.at[:, pl.ds(0, L)])

        def copy_row(r, off):
            len_s = get_len(c * CH + r)
            nsl_r = (len_s + 15) >> 4

            def copy_slice(j, _):
                f = off + j * LANES
                idx_m[f >> 7, pl.ds(f & (NIDX - 1), LANES)] = (
                    idx_raw[r, pl.ds(j * LANES, LANES)])
                return 0

            lax.fori_loop(0, nsl_r, copy_slice, 0)
            return off + (nsl_r << 4)

        total = lax.fori_loop(0, CH, copy_row, 0)
        ndma = (total + NIDX - 1) >> 7

        def fire(j, _):
            pltpu.async_copy(table_hbm.at[idx_m.at[j]],
                             gath.at[pl.ds(j * NIDX, NIDX)], sem)
            return 0

        lax.fori_loop(0, ndma, fire, 0)
        return ndma

    def finish(c, ndma, idx_m, gath, sem):
        base = wbase + c * CH

        def drain(j, _):
            pltpu.make_async_copy(table_hbm.at[idx_m.at[j]],
                                  gath.at[pl.ds(j * NIDX, NIDX)], sem).wait()
            return 0

        lax.fori_loop(0, ndma, drain, 0)

        def fin_row(r, off):
            len_s = get_len(c * CH + r)
            nfull = len_s >> 4
            nsl_r = (len_s + 15) >> 4

            def acc_slice(j, carry):
                a0, a1, b0, b1 = carry
                g = off + j * LANES
                for q in range(0, LANES, 2):
                    a0 = a0 + gath[g + q, pl.ds(0, LANES)]
                    a1 = a1 + gath[g + q, pl.ds(LANES, LANES)]
                    b0 = b0 + gath[g + q + 1, pl.ds(0, LANES)]
                    b1 = b1 + gath[g + q + 1, pl.ds(LANES, LANES)]
                return (a0, a1, b0, b1)

            zero = jnp.zeros((LANES,), jnp.float32)
            a0, a1, b0, b1 = lax.fori_loop(0, nfull, acc_slice,
                                           (zero, zero, zero, zero))
            a0 = a0 + b0
            a1 = a1 + b1

            def acc_tail(p, carry):
                t0, t1 = carry
                g = off + p
                return (t0 + gath[g, pl.ds(0, LANES)],
                        t1 + gath[g, pl.ds(LANES, LANES)])

            a0, a1 = lax.fori_loop(nfull << 4, len_s, acc_tail, (a0, a1))

            def z_body(j, zc):
                f = off + j * LANES
                sel = idx_m[f >> 7, pl.ds(f & (NIDX - 1), LANES)]
                pos = j * LANES + iota
                return zc + jnp.where((pos < len_s) & (sel == 0), 1, 0)

            zc = lax.fori_loop(0, nsl_r, z_body, zero16i)
            zf = jnp.broadcast_to(jnp.sum(zc), (LANES,)).astype(jnp.float32)

            den = jnp.broadcast_to(jnp.maximum(len_s, 1), (LANES,))
            inv = jnp.ones((LANES,), jnp.float32) / den.astype(jnp.float32)
            out_v[r, pl.ds(0, LANES)] = (a0 - zf * r0a) * inv
            out_v[r, pl.ds(LANES, LANES)] = (a1 - zf * r0b) * inv
            return off + (nsl_r << 4)

        lax.fori_loop(0, CH, fin_row, 0)
        pltpu.sync_copy(out_v, out_hbm.at[pl.ds(base, CH)])

    prev0 = prep(0, idx_m0, gath0, sem0)

    def body(c, prev):
        def odd():
            nd = prep(c, idx_m1, gath1, sem1)
            finish(c - 1, prev, idx_m0, gath0, sem0)
            return nd

        def even():
            nd = prep(c, idx_m0, gath0, sem0)
            finish(c - 1, prev, idx_m1, gath1, sem1)
            return nd

        return lax.cond((c & 1) == 1, odd, even)

    last = lax.fori_loop(1, NCHUNK, body, prev0)
    finish(NCHUNK - 1, last, idx_m1, gath1, sem1)


@jax.jit
def _run(stacks, stack_lens, table):
    mesh = plsc.VectorSubcoreMesh(core_axis_name="c", subcore_axis_name="s")
    f = pl.kernel(
        _body,
        out_type=jax.ShapeDtypeStruct((B, DIM), jnp.float32),
        mesh=mesh,
        scratch_types=[
            pltpu.VMEM((CH, LP), jnp.int32),
            pltpu.VMEM((NBLK, NIDX), jnp.int32),
            pltpu.VMEM((NBLK, NIDX), jnp.int32),
            pltpu.VMEM((GI, DIM), jnp.float32),
            pltpu.VMEM((GI, DIM), jnp.float32),
            pltpu.VMEM((RPW,), jnp.int32),
            pltpu.VMEM((CH, DIM), jnp.float32),
            pltpu.VMEM((1, DIM), jnp.float32),
            pltpu.SemaphoreType.DMA,
            pltpu.SemaphoreType.DMA,
        ],
        compiler_params=pltpu.CompilerParams(use_tc_tiling_on_sc=False,
                                             needs_layout_passes=False),
    )
    return f(stacks, stack_lens, table)


def kernel(stacks, stack_lens, table):
    return _run(stacks.astype(jnp.int32), stack_lens.astype(jnp.int32), table)
